# Initial kernel scaffold; baseline (speedup 1.0000x reference)
import jax
import jax.numpy as jnp
from jax.experimental import pallas as pl


def kernel(inputs, table):
    raise NotImplementedError("write your pallas kernel here")


# speedup vs baseline: 1.2852x; 1.2852x over previous
import functools

import jax
import jax.numpy as jnp
from jax import lax
from jax.experimental import pallas as pl
from jax.experimental.pallas import tpu as pltpu
from jax.experimental.pallas import tpu_sc as plsc

NUM_FIELDS = 26
VOCAB_PER_FIELD = 100000
EMBED_DIM = 128
BATCH = 4096

NC, NS, L = 2, 16, 16
NW = NC * NS
N_FLAT = BATCH * NUM_FIELDS
PER_W = N_FLAT // NW
CHUNK = 128
N_CHUNK = PER_W // CHUNK


@functools.partial(
    pl.kernel,
    out_type=jax.ShapeDtypeStruct((N_FLAT, EMBED_DIM), jnp.float32),
    mesh=plsc.VectorSubcoreMesh(core_axis_name="c", subcore_axis_name="s"),
    scratch_types=[
        pltpu.VMEM((PER_W,), jnp.int32),
        pltpu.VMEM((CHUNK, EMBED_DIM), jnp.float32),
        pltpu.VMEM((CHUNK, EMBED_DIM), jnp.float32),
        pltpu.SemaphoreType.DMA,
        pltpu.SemaphoreType.DMA,
        pltpu.SemaphoreType.DMA,
        pltpu.SemaphoreType.DMA,
    ],
)
def _sok_gather(idx_hbm, table_hbm, out_hbm, idx_v, buf0, buf1, g0, g1, s0, s1):
    wid = lax.axis_index("s") * NC + lax.axis_index("c")
    base = wid * PER_W

    pltpu.sync_copy(idx_hbm.at[pl.ds(base, PER_W)], idx_v)
    iota = lax.iota(jnp.int32, L)

    @pl.loop(0, PER_W // L)
    def _fuse(t):
        pos = base + t * L + iota
        off = lax.rem(pos, NUM_FIELDS) * VOCAB_PER_FIELD
        idx_v[pl.ds(t * L, L)] = idx_v[pl.ds(t * L, L)] + off

    bufs = (buf0, buf1)
    gsem = (g0, g1)
    ssem = (s0, s1)

    def gather(j, b):
        pltpu.async_copy(table_hbm.at[idx_v.at[pl.ds(j * CHUNK, CHUNK)]],
                         bufs[b], gsem[b])

    def wait_gather(b):
        pltpu.make_async_copy(out_hbm.at[pl.ds(0, CHUNK)], bufs[b], gsem[b]).wait()

    def store(j, b):
        pltpu.async_copy(bufs[b], out_hbm.at[pl.ds((base + j * CHUNK), CHUNK)],
                         ssem[b])

    def wait_store(b):
        pltpu.make_async_copy(bufs[b], out_hbm.at[pl.ds(0, CHUNK)], ssem[b]).wait()

    gather(0, 0)
    gather(1, 1)

    @pl.loop(0, N_CHUNK - 2, step=2)
    def _main(j0):
        for b in range(2):
            j = j0 + b
            wait_gather(b)
            store(j, b)
            wait_store(b)
            gather(j + 2, b)

    for b in range(2):
        wait_gather(b)
        store(N_CHUNK - 2 + b, b)
        wait_store(b)


def kernel(inputs, table):
    flat_ids = inputs.reshape(-1)
    out = _sok_gather(flat_ids, table)
    return out.reshape(BATCH, NUM_FIELDS, EMBED_DIM)


# speedup vs baseline: 1.3034x; 1.0142x over previous
import functools

import jax
import jax.numpy as jnp
from jax import lax
from jax.experimental import pallas as pl
from jax.experimental.pallas import tpu as pltpu
from jax.experimental.pallas import tpu_sc as plsc

NUM_FIELDS = 26
VOCAB_PER_FIELD = 100000
EMBED_DIM = 128
BATCH = 4096

NC, NS, L = 2, 16, 16
NW = NC * NS
N_FLAT = BATCH * NUM_FIELDS
PER_W = N_FLAT // NW
CHUNK = 104
N_CHUNK = PER_W // CHUNK
NBUF = 4


@functools.partial(
    pl.kernel,
    out_type=jax.ShapeDtypeStruct((N_FLAT, EMBED_DIM), jnp.float32),
    mesh=plsc.VectorSubcoreMesh(core_axis_name="c", subcore_axis_name="s"),
    scratch_types=[
        pltpu.VMEM((PER_W,), jnp.int32),
    ] + [pltpu.VMEM((CHUNK, EMBED_DIM), jnp.float32) for _ in range(NBUF)]
      + [pltpu.SemaphoreType.DMA for _ in range(2 * NBUF)],
)
def _sok_gather(idx_hbm, table_hbm, out_hbm, idx_v, *rest):
    bufs = rest[:NBUF]
    gsem = rest[NBUF:2 * NBUF]
    ssem = rest[2 * NBUF:]
    wid = lax.axis_index("s") * NC + lax.axis_index("c")
    base = wid * PER_W

    pltpu.sync_copy(idx_hbm.at[pl.ds(base, PER_W)], idx_v)
    iota = lax.iota(jnp.int32, L)

    @pl.loop(0, PER_W // L, unroll=8)
    def _fuse(t):
        pos = base + t * L + iota
        off = lax.rem(pos, NUM_FIELDS) * VOCAB_PER_FIELD
        idx_v[pl.ds(t * L, L)] = idx_v[pl.ds(t * L, L)] + off

    def gather(j, b):
        pltpu.async_copy(table_hbm.at[idx_v.at[pl.ds(j * CHUNK, CHUNK)]],
                         bufs[b], gsem[b])

    def wait_gather(b):
        pltpu.make_async_copy(out_hbm.at[pl.ds(0, CHUNK)], bufs[b], gsem[b]).wait()

    def store(j, b):
        pltpu.async_copy(bufs[b], out_hbm.at[pl.ds((base + j * CHUNK), CHUNK)],
                         ssem[b])

    def wait_store(b):
        pltpu.make_async_copy(bufs[b], out_hbm.at[pl.ds(0, CHUNK)], ssem[b]).wait()

    for b in range(NBUF):
        gather(b, b)

    @pl.loop(0, N_CHUNK - NBUF, step=NBUF)
    def _main(j0):
        for b in range(NBUF):
            j = j0 + b
            wait_gather(b)
            store(j, b)
            wait_store(b)
            gather(j + NBUF, b)

    for b in range(NBUF):
        wait_gather(b)
        store(N_CHUNK - NBUF + b, b)
        wait_store(b)


def kernel(inputs, table):
    flat_ids = inputs.reshape(-1)
    out = _sok_gather(flat_ids, table)
    return out.reshape(BATCH, NUM_FIELDS, EMBED_DIM)
